# Initial kernel scaffold; baseline (speedup 1.0000x reference)
import jax
import jax.numpy as jnp
from jax.experimental import pallas as pl


def kernel(payload_tensor, W, b):
    raise NotImplementedError("write your pallas kernel here")


# speedup vs baseline: 6.6532x; 6.6532x over previous
import jax
import jax.numpy as jnp
from jax.experimental import pallas as pl

_TOKENS = 32768
_D = 768
_E = 64
_TILE = 1024


def _gate_body(x_ref, w_ref, b_ref, o_ref):
    x = x_ref[...]
    w = w_ref[...]
    bias = b_ref[...]
    logits = jax.lax.dot_general(
        x, w, (((1,), (1,)), ((), ())),
        preferred_element_type=jnp.float32) + bias
    col = jax.lax.broadcasted_iota(jnp.int32, logits.shape, 1)
    m1 = jnp.max(logits, axis=1, keepdims=True)
    i1 = jnp.min(jnp.where(logits == m1, col, _E), axis=1, keepdims=True)
    l2 = jnp.where(col == i1, -jnp.inf, logits)
    m2 = jnp.max(l2, axis=1, keepdims=True)
    i2 = jnp.min(jnp.where(l2 == m2, col, _E), axis=1, keepdims=True)
    s = jnp.exp(m2 - m1)
    denom = 1.0 + s
    w1 = 1.0 / denom
    w2 = s / denom
    o_ref[...] = jnp.where(col == i1, w1, 0.0) + jnp.where(col == i2, w2, 0.0)


def kernel(payload_tensor, W, b):
    b2 = b.reshape(1, _E)
    grid = (_TOKENS // _TILE,)
    return pl.pallas_call(
        _gate_body,
        grid=grid,
        in_specs=[
            pl.BlockSpec((_TILE, _D), lambda i: (i, 0)),
            pl.BlockSpec((_E, _D), lambda i: (0, 0)),
            pl.BlockSpec((1, _E), lambda i: (0, 0)),
        ],
        out_specs=pl.BlockSpec((_TILE, _E), lambda i: (i, 0)),
        out_shape=jax.ShapeDtypeStruct((_TOKENS, _E), jnp.float32),
    )(payload_tensor, W, b2)


# speedup vs baseline: 7.5701x; 1.1378x over previous
import jax
import jax.numpy as jnp
from jax.experimental import pallas as pl

_TOKENS = 32768
_D = 768
_E = 64
_TILE = 2048


def _gate_body(x_ref, w_ref, b_ref, o_ref):
    x = x_ref[...]
    w = w_ref[...]
    bias = b_ref[...]
    logits = jax.lax.dot_general(
        x, w, (((1,), (1,)), ((), ())),
        preferred_element_type=jnp.float32) + bias
    col = jax.lax.broadcasted_iota(jnp.int32, logits.shape, 1)
    m1 = jnp.max(logits, axis=1, keepdims=True)
    i1 = jnp.min(jnp.where(logits == m1, col, _E), axis=1, keepdims=True)
    l2 = jnp.where(col == i1, -jnp.inf, logits)
    m2 = jnp.max(l2, axis=1, keepdims=True)
    i2 = jnp.min(jnp.where(l2 == m2, col, _E), axis=1, keepdims=True)
    s = jnp.exp(m2 - m1)
    denom = 1.0 + s
    w1 = 1.0 / denom
    w2 = s / denom
    o_ref[...] = jnp.where(col == i1, w1, 0.0) + jnp.where(col == i2, w2, 0.0)


def kernel(payload_tensor, W, b):
    b2 = b.reshape(1, _E)
    grid = (_TOKENS // _TILE,)
    return pl.pallas_call(
        _gate_body,
        grid=grid,
        in_specs=[
            pl.BlockSpec((_TILE, _D), lambda i: (i, 0)),
            pl.BlockSpec((_E, _D), lambda i: (0, 0)),
            pl.BlockSpec((1, _E), lambda i: (0, 0)),
        ],
        out_specs=pl.BlockSpec((_TILE, _E), lambda i: (i, 0)),
        out_shape=jax.ShapeDtypeStruct((_TOKENS, _E), jnp.float32),
    )(payload_tensor, W, b2)


# speedup vs baseline: 8.2877x; 1.0948x over previous
import jax
import jax.numpy as jnp
from jax.experimental import pallas as pl

_TOKENS = 32768
_D = 768
_E = 64
_TILE = 4096


def _gate_body(x_ref, w_ref, b_ref, o_ref):
    x = x_ref[...]
    w = w_ref[...]
    bias = b_ref[...]
    logits = jax.lax.dot_general(
        x, w, (((1,), (1,)), ((), ())),
        preferred_element_type=jnp.float32) + bias
    col = jax.lax.broadcasted_iota(jnp.int32, logits.shape, 1)
    m1 = jnp.max(logits, axis=1, keepdims=True)
    i1 = jnp.min(jnp.where(logits == m1, col, _E), axis=1, keepdims=True)
    l2 = jnp.where(col == i1, -jnp.inf, logits)
    m2 = jnp.max(l2, axis=1, keepdims=True)
    i2 = jnp.min(jnp.where(l2 == m2, col, _E), axis=1, keepdims=True)
    s = jnp.exp(m2 - m1)
    denom = 1.0 + s
    w1 = 1.0 / denom
    w2 = s / denom
    o_ref[...] = jnp.where(col == i1, w1, 0.0) + jnp.where(col == i2, w2, 0.0)


def kernel(payload_tensor, W, b):
    b2 = b.reshape(1, _E)
    grid = (_TOKENS // _TILE,)
    return pl.pallas_call(
        _gate_body,
        grid=grid,
        in_specs=[
            pl.BlockSpec((_TILE, _D), lambda i: (i, 0)),
            pl.BlockSpec((_E, _D), lambda i: (0, 0)),
            pl.BlockSpec((1, _E), lambda i: (0, 0)),
        ],
        out_specs=pl.BlockSpec((_TILE, _E), lambda i: (i, 0)),
        out_shape=jax.ShapeDtypeStruct((_TOKENS, _E), jnp.float32),
    )(payload_tensor, W, b2)
